# Initial kernel scaffold; baseline (speedup 1.0000x reference)
import jax
import jax.numpy as jnp
from jax.experimental import pallas as pl


def kernel(token_ids, weight):
    raise NotImplementedError("write your pallas kernel here")


# speedup vs baseline: 1.1078x; 1.1078x over previous
import functools

import jax
import jax.numpy as jnp
from jax import lax
from jax.experimental import pallas as pl
from jax.experimental.pallas import tpu as pltpu
from jax.experimental.pallas import tpu_sc as plsc

_NUM_CORES = 2
_NUM_SUBCORES = 16
_NUM_WORKERS = _NUM_CORES * _NUM_SUBCORES

_CHUNK = 2560


def _embedding_lookup(idx, weight, out_rows, out_dim):
  b_per_w = out_rows // _NUM_WORKERS
  n_chunks = b_per_w // _CHUNK
  mesh = plsc.VectorSubcoreMesh(core_axis_name="c", subcore_axis_name="s")

  @functools.partial(
      pl.kernel,
      out_type=jax.ShapeDtypeStruct((out_rows, out_dim), jnp.float32),
      mesh=mesh,
      scratch_types=[
          pltpu.VMEM((_CHUNK,), jnp.int32),
          pltpu.VMEM((_CHUNK, out_dim), jnp.float32),
          pltpu.SemaphoreType.DMA,
      ],
      compiler_params=pltpu.CompilerParams(use_tc_tiling_on_sc=False),
  )
  def k(idx_hbm, table_hbm, out_hbm, idx_v, rows_v, sem):
    wid = lax.axis_index("s") * _NUM_CORES + lax.axis_index("c")

    def body(i, carry):
      base = wid * b_per_w + i * _CHUNK
      pltpu.sync_copy(idx_hbm.at[pl.ds(base, _CHUNK)], idx_v)
      pltpu.async_copy(table_hbm.at[idx_v], rows_v, sem).wait()
      pltpu.sync_copy(rows_v, out_hbm.at[pl.ds(base, _CHUNK)])
      return carry

    lax.fori_loop(0, n_chunks, body, 0)

  return k(idx, weight)


def kernel(token_ids, weight):
  b0, s = token_ids.shape
  v, d = weight.shape
  flat = token_ids.reshape(b0 * s).astype(jnp.int32)
  out = _embedding_lookup(flat, weight, b0 * s, d)
  return out.reshape(b0, s, d)


# speedup vs baseline: 1.1121x; 1.0039x over previous
import functools

import jax
import jax.numpy as jnp
from jax import lax
from jax.experimental import pallas as pl
from jax.experimental.pallas import tpu as pltpu
from jax.experimental.pallas import tpu_sc as plsc

_NUM_CORES = 2
_NUM_SUBCORES = 16
_NUM_WORKERS = _NUM_CORES * _NUM_SUBCORES

_CHUNK = 1600


def _embedding_lookup(idx, weight, out_rows, out_dim):
  b_per_w = out_rows // _NUM_WORKERS
  n_chunks = b_per_w // _CHUNK
  mesh = plsc.VectorSubcoreMesh(core_axis_name="c", subcore_axis_name="s")

  @functools.partial(
      pl.kernel,
      out_type=jax.ShapeDtypeStruct((out_rows, out_dim), jnp.float32),
      mesh=mesh,
      scratch_types=[
          pltpu.VMEM((2, _CHUNK), jnp.int32),
          pltpu.VMEM((2, _CHUNK, out_dim), jnp.float32),
          pltpu.SemaphoreType.DMA((2,)),
          pltpu.SemaphoreType.DMA((2,)),
      ],
      compiler_params=pltpu.CompilerParams(use_tc_tiling_on_sc=False),
  )
  def k(idx_hbm, table_hbm, out_hbm, idx_v, rows_v, gsem, osem):
    wid = lax.axis_index("s") * _NUM_CORES + lax.axis_index("c")
    base0 = wid * b_per_w

    gathers = [None] * n_chunks
    stores = [None] * n_chunks

    pltpu.sync_copy(idx_hbm.at[pl.ds(base0, _CHUNK)], idx_v.at[0])
    gathers[0] = pltpu.async_copy(
        table_hbm.at[idx_v.at[0]], rows_v.at[0], gsem.at[0])

    for i in range(n_chunks):
      b = i % 2
      nb = (i + 1) % 2
      if i + 1 < n_chunks:
        pltpu.sync_copy(
            idx_hbm.at[pl.ds(base0 + (i + 1) * _CHUNK, _CHUNK)], idx_v.at[nb])
        if i >= 1:
          stores[i - 1].wait()
        gathers[i + 1] = pltpu.async_copy(
            table_hbm.at[idx_v.at[nb]], rows_v.at[nb], gsem.at[nb])
      gathers[i].wait()
      stores[i] = pltpu.async_copy(
          rows_v.at[b], out_hbm.at[pl.ds(base0 + i * _CHUNK, _CHUNK)],
          osem.at[b])

    stores[n_chunks - 2].wait()
    stores[n_chunks - 1].wait()

  return k(idx, weight)


def kernel(token_ids, weight):
  b0, s = token_ids.shape
  v, d = weight.shape
  flat = token_ids.reshape(b0 * s).astype(jnp.int32)
  out = _embedding_lookup(flat, weight, b0 * s, d)
  return out.reshape(b0, s, d)


# speedup vs baseline: 1.8019x; 1.6202x over previous
import functools

import jax
import jax.numpy as jnp
from jax import lax
from jax.experimental import pallas as pl
from jax.experimental.pallas import tpu as pltpu
from jax.experimental.pallas import tpu_sc as plsc

_NUM_CORES = 2
_NUM_SUBCORES = 16
_NUM_WORKERS = _NUM_CORES * _NUM_SUBCORES

_CHUNK = 1600


def _embedding_lookup(ids, weight, n_rows, seq):
  _, dim = weight.shape
  n_ids = n_rows * seq
  b_per_w = n_ids // _NUM_WORKERS
  n_chunks = b_per_w // _CHUNK
  mesh = plsc.VectorSubcoreMesh(core_axis_name="c", subcore_axis_name="s")

  @functools.partial(
      pl.kernel,
      out_type=jax.ShapeDtypeStruct((n_rows, seq, dim), jnp.float32),
      mesh=mesh,
      scratch_types=[
          pltpu.VMEM((2, _CHUNK), jnp.int32),
          pltpu.VMEM((2, _CHUNK, dim), jnp.float32),
          pltpu.SemaphoreType.DMA((2,)),
          pltpu.SemaphoreType.DMA((2,)),
      ],
      compiler_params=pltpu.CompilerParams(use_tc_tiling_on_sc=False),
  )
  def k(idx_hbm, table_hbm, out_hbm, idx_v, rows_v, gsem, osem):
    idx_flat = idx_hbm
    wid = lax.axis_index("s") * _NUM_CORES + lax.axis_index("c")
    base0 = wid * b_per_w

    gathers = [None] * n_chunks
    stores = [None] * n_chunks

    pltpu.sync_copy(idx_flat.at[pl.ds(base0, _CHUNK)], idx_v.at[0])
    gathers[0] = pltpu.async_copy(
        table_hbm.at[idx_v.at[0]], rows_v.at[0], gsem.at[0])

    for i in range(n_chunks):
      b = i % 2
      nb = (i + 1) % 2
      if i + 1 < n_chunks:
        pltpu.sync_copy(
            idx_flat.at[pl.ds(base0 + (i + 1) * _CHUNK, _CHUNK)],
            idx_v.at[nb])
        if i >= 1:
          for st in stores[i - 1]:
            st.wait()
        gathers[i + 1] = pltpu.async_copy(
            table_hbm.at[idx_v.at[nb]], rows_v.at[nb], gsem.at[nb])
      gathers[i].wait()
      rows_per_chunk = _CHUNK // seq
      row0 = wid * (rows_per_chunk * n_chunks) + i * rows_per_chunk
      stores[i] = [
          pltpu.async_copy(
              rows_v.at[b, pl.ds(j * seq, seq)], out_hbm.at[row0 + j],
              osem.at[b])
          for j in range(rows_per_chunk)
      ]

    for st in stores[n_chunks - 2]:
      st.wait()
    for st in stores[n_chunks - 1]:
      st.wait()

  return k(ids, weight)


def kernel(token_ids, weight):
  n_rows, seq = token_ids.shape
  flat = token_ids.reshape(n_rows * seq).astype(jnp.int32)
  return _embedding_lookup(flat, weight, n_rows, seq)
